# Initial kernel scaffold; baseline (speedup 1.0000x reference)
import jax
import jax.numpy as jnp
from jax.experimental import pallas as pl


def kernel(boxes, scores):
    raise NotImplementedError("write your pallas kernel here")


# speedup vs baseline: 104.2100x; 104.2100x over previous
import functools

import jax
import jax.numpy as jnp
from jax.experimental import pallas as pl

_SIGMA = 0.5
_SCORE_THR = 0.05
_ROWS = 8
_COLS = 640
_PAD_N = _ROWS * _COLS


def _soft_nms_body(x1_ref, y1_ref, x2_ref, y2_ref, s_ref, out_ref):
    x1 = x1_ref[...]
    y1 = y1_ref[...]
    x2 = x2_ref[...]
    y2 = y2_ref[...]
    area = (x2 - x1) * (y2 - y1)

    row = jax.lax.broadcasted_iota(jnp.int32, (_ROWS, _COLS), 0)
    col = jax.lax.broadcasted_iota(jnp.int32, (_ROWS, _COLS), 1)
    iiota = row * _COLS + col

    w0 = s_ref[...]
    out0 = jnp.zeros((_ROWS, _COLS), jnp.float32)

    def cond(carry):
        _, _, maxv = carry
        return maxv > _SCORE_THR

    def body(carry):
        w, out, maxv = carry
        mask = w == maxv
        idx = jnp.min(jnp.where(mask, iiota, jnp.int32(2**30)))
        onehot = iiota == idx
        out = jnp.where(onehot, maxv, out)
        bx1 = jnp.sum(jnp.where(onehot, x1, 0.0))
        by1 = jnp.sum(jnp.where(onehot, y1, 0.0))
        bx2 = jnp.sum(jnp.where(onehot, x2, 0.0))
        by2 = jnp.sum(jnp.where(onehot, y2, 0.0))
        iw = jnp.clip(jnp.minimum(bx2, x2) - jnp.maximum(bx1, x1), 0.0)
        ih = jnp.clip(jnp.minimum(by2, y2) - jnp.maximum(by1, y1), 0.0)
        inter = iw * ih
        barea = (bx2 - bx1) * (by2 - by1)
        iou = inter / (barea + area - inter + 1e-6)
        weight = jnp.exp(-(iou * iou) / _SIGMA)
        w = jnp.where(onehot, -jnp.inf, w * weight)
        return w, out, jnp.max(w)

    _, out, _ = jax.lax.while_loop(cond, body, (w0, out0, jnp.max(w0)))
    out_ref[...] = jnp.where(out > _SCORE_THR, out, 0.0)


@functools.partial(jax.jit, static_argnames=())
def kernel(boxes, scores):
    n = boxes.shape[0]
    pad = _PAD_N - n

    def shape(v, fill):
        return jnp.pad(v, (0, pad), constant_values=fill).reshape(_ROWS, _COLS)

    x1 = shape(boxes[:, 0], 0.0)
    y1 = shape(boxes[:, 1], 0.0)
    x2 = shape(boxes[:, 2], 0.0)
    y2 = shape(boxes[:, 3], 0.0)
    s = shape(scores, -jnp.inf)

    out = pl.pallas_call(
        _soft_nms_body,
        out_shape=jax.ShapeDtypeStruct((_ROWS, _COLS), jnp.float32),
    )(x1, y1, x2, y2, s)
    return out.reshape(-1)[:n]


# speedup vs baseline: 192.7097x; 1.8492x over previous
import functools

import jax
import jax.numpy as jnp
from jax.experimental import pallas as pl
from jax.experimental.pallas import tpu as pltpu

_SIGMA = 0.5
_SCORE_THR = 0.05
_ROWS = 8
_COLS = 640
_PAD_N = _ROWS * _COLS


def _soft_nms_body(cs_ref, x1_ref, y1_ref, x2_ref, y2_ref, s_ref, out_ref):
    x1 = x1_ref[...]
    y1 = y1_ref[...]
    x2 = x2_ref[...]
    y2 = y2_ref[...]
    area = (x2 - x1) * (y2 - y1)

    row = jax.lax.broadcasted_iota(jnp.int32, (_ROWS, _COLS), 0)
    col = jax.lax.broadcasted_iota(jnp.int32, (_ROWS, _COLS), 1)
    iiota = row * _COLS + col

    w0 = s_ref[...]
    out0 = jnp.zeros((_ROWS, _COLS), jnp.float32)

    def cond(carry):
        _, _, maxv, _ = carry
        return maxv > _SCORE_THR

    def body(carry):
        w, out, maxv, m = carry
        onehot = iiota == m
        out = jnp.where(onehot, maxv, out)
        bx1 = cs_ref[0, m]
        by1 = cs_ref[1, m]
        bx2 = cs_ref[2, m]
        by2 = cs_ref[3, m]
        iw = jnp.clip(jnp.minimum(bx2, x2) - jnp.maximum(bx1, x1), 0.0)
        ih = jnp.clip(jnp.minimum(by2, y2) - jnp.maximum(by1, y1), 0.0)
        inter = iw * ih
        barea = (bx2 - bx1) * (by2 - by1)
        iou = inter / (barea + area - inter + 1e-6)
        weight = jnp.exp(-(iou * iou) / _SIGMA)
        w = jnp.where(onehot, -jnp.inf, w * weight)
        return w, out, jnp.max(w), jnp.argmax(w).astype(jnp.int32)

    init = (w0, out0, jnp.max(w0), jnp.argmax(w0).astype(jnp.int32))
    _, out, _, _ = jax.lax.while_loop(cond, body, init)
    out_ref[...] = jnp.where(out > _SCORE_THR, out, 0.0)


@functools.partial(jax.jit, static_argnames=())
def kernel(boxes, scores):
    n = boxes.shape[0]
    pad = _PAD_N - n

    def shape(v, fill):
        return jnp.pad(v, (0, pad), constant_values=fill).reshape(_ROWS, _COLS)

    x1 = shape(boxes[:, 0], 0.0)
    y1 = shape(boxes[:, 1], 0.0)
    x2 = shape(boxes[:, 2], 0.0)
    y2 = shape(boxes[:, 3], 0.0)
    s = shape(scores, -jnp.inf)
    coords_smem = jnp.pad(boxes.T, ((0, 0), (0, pad)))

    out = pl.pallas_call(
        _soft_nms_body,
        in_specs=[
            pl.BlockSpec(memory_space=pltpu.SMEM),
            pl.BlockSpec(memory_space=pltpu.VMEM),
            pl.BlockSpec(memory_space=pltpu.VMEM),
            pl.BlockSpec(memory_space=pltpu.VMEM),
            pl.BlockSpec(memory_space=pltpu.VMEM),
            pl.BlockSpec(memory_space=pltpu.VMEM),
        ],
        out_shape=jax.ShapeDtypeStruct((_ROWS, _COLS), jnp.float32),
    )(coords_smem, x1, y1, x2, y2, s)
    return out.reshape(-1)[:n]
